# Initial kernel scaffold; baseline (speedup 1.0000x reference)
import jax
import jax.numpy as jnp
from jax.experimental import pallas as pl


def kernel(xyz, new_xyz, features):
    raise NotImplementedError("write your pallas kernel here")


# speedup vs baseline: 5.2077x; 5.2077x over previous
import jax
import jax.numpy as jnp
from jax import lax
from jax.experimental import pallas as pl
from jax.experimental.pallas import tpu as pltpu
from jax.experimental.pallas import tpu_sc as plsc

B = 4
N = 8192
S = 1024
NSAMPLE = 32
C = 128
NCH = C + 3
RAD2 = 0.2 * 0.2

NC = 2
NSUB = 16
L = 16
NW = NC * NSUB
WPB = NW // B
QPW = S // WPB
CPW = (NCH + WPB - 1) // WPB


def _mesh():
    return plsc.VectorSubcoreMesh(
        core_axis_name="c", subcore_axis_name="s", num_cores=NC, num_subcores=NSUB
    )


def _bq_body(xt, nxt, idx_out, pb, qb, cand, obuf):
    w = lax.axis_index("s") * NC + lax.axis_index("c")
    b = w // WPB
    q0 = (w % WPB) * QPW
    pltpu.sync_copy(xt.at[b], pb)
    pltpu.sync_copy(nxt.at[b], qb)
    lanes = lax.iota(jnp.int32, L)

    def per_group(qg, carry):
        qoff = pl.multiple_of(q0 + qg * L, L)
        qxv = qb[0, pl.ds(qoff, L)]
        qyv = qb[1, pl.ds(qoff, L)]
        qzv = qb[2, pl.ds(qoff, L)]
        for i in range(L):
            qx, qy, qz = qxv[i], qyv[i], qzv[i]

            def step(t, cnt):
                base = pl.multiple_of(t * L, L)
                dx = pb[0, pl.ds(base, L)] - qx
                dy = pb[1, pl.ds(base, L)] - qy
                dz = pb[2, pl.ds(base, L)] - qz
                d = dx * dx + dy * dy + dz * dz
                m = d < RAD2
                hits = jnp.sum(m.astype(jnp.int32))
                pred = jnp.logical_and(hits > 0, cnt < NSAMPLE)

                @pl.when(pred)
                def _():
                    slots = cnt + plsc.cumsum(m.astype(jnp.int32)) - 1
                    mm = jnp.logical_and(m, slots < NSAMPLE)
                    plsc.store_scatter(cand, [slots], base + lanes, mask=mm)

                return jnp.where(pred, cnt + hits, cnt)

            cnt = lax.fori_loop(0, N // L, step, jnp.int32(0))
            first = cand[pl.ds(0, L)][0]
            fill = jnp.where(cnt > 0, first, 0)
            for h in range(NSAMPLE // L):
                slots = h * L + lanes
                cur = cand[pl.ds(h * L, L)]
                obuf[qg * L + i, pl.ds(h * L, L)] = jnp.where(slots < cnt, cur, fill)
        return carry

    lax.fori_loop(0, QPW // L, per_group, 0)
    pltpu.sync_copy(obuf, idx_out.at[b, pl.ds(q0, QPW)])


def _grp_body(tbl, nxt, idxf, out, ib, rb, ob, cb):
    w = lax.axis_index("s") * NC + lax.axis_index("c")
    b = w // WPB
    g = w % WPB
    pltpu.sync_copy(idxf.at[b], ib)
    pltpu.sync_copy(nxt.at[b], cb)
    nv = S * NSAMPLE // L

    def per_chan(j, carry):
        ch = g + WPB * j

        @pl.when(ch < NCH)
        def _():
            pltpu.sync_copy(tbl.at[b * NCH + ch], rb)

            zv = jnp.zeros((L,), jnp.int32)

            def gstep(v, c2):
                off = pl.multiple_of(v * L, L)
                ivec = ib[0, pl.ds(off, L)]
                ob[0, pl.ds(off, L)] = plsc.load_gather(rb, [zv, ivec])
                return c2

            lax.fori_loop(0, nv, gstep, 0)

            @pl.when(ch < 3)
            def _():
                crow = jnp.where(ch < 3, ch, 0)

                def sgrp(sg, c3):
                    soff = pl.multiple_of(sg * L, L)
                    ctrv = cb[crow, pl.ds(soff, L)]
                    for i in range(L):
                        ctr = ctrv[i]
                        p = (sg * L + i) * NSAMPLE
                        for h in range(NSAMPLE // L):
                            off = pl.multiple_of(p + h * L, L)
                            ob[0, pl.ds(off, L)] = ob[0, pl.ds(off, L)] - ctr
                    return c3

                lax.fori_loop(0, S // L, sgrp, 0)

            pltpu.sync_copy(ob, out.at[b * NCH + ch])

        return carry

    lax.fori_loop(0, CPW, per_chan, 0)


def _ball_query(xt, nxt, interpret=False):
    return pl.kernel(
        _bq_body,
        out_type=jax.ShapeDtypeStruct((B, S, NSAMPLE), jnp.int32),
        mesh=_mesh(),
        scratch_types=[
            pltpu.VMEM((3, N), jnp.float32),
            pltpu.VMEM((3, S), jnp.float32),
            pltpu.VMEM((NSAMPLE,), jnp.int32),
            pltpu.VMEM((QPW, NSAMPLE), jnp.int32),
        ],
        compiler_params=pltpu.CompilerParams(needs_layout_passes=False),
        interpret=interpret,
    )(xt, nxt)


def _group(tbl, nxt, idxf, interpret=False):
    return pl.kernel(
        _grp_body,
        out_type=jax.ShapeDtypeStruct((B * NCH, 1, S * NSAMPLE), jnp.float32),
        mesh=_mesh(),
        scratch_types=[
            pltpu.VMEM((1, S * NSAMPLE), jnp.int32),
            pltpu.VMEM((1, N), jnp.float32),
            pltpu.VMEM((1, S * NSAMPLE), jnp.float32),
            pltpu.VMEM((3, S), jnp.float32),
        ],
        compiler_params=pltpu.CompilerParams(needs_layout_passes=False),
        interpret=interpret,
    )(tbl, nxt, idxf)


def kernel(xyz, new_xyz, features, interpret=False):
    xt = jnp.transpose(xyz, (0, 2, 1))
    nxt = jnp.transpose(new_xyz, (0, 2, 1))
    idx = _ball_query(xt, nxt, interpret=interpret)
    idxf = idx.reshape(B, 1, S * NSAMPLE)
    tbl = jnp.concatenate([xt, features], axis=1).reshape(B * NCH, 1, N)
    out = _group(tbl, nxt, idxf, interpret=interpret)
    return out.reshape(B, NCH, S, NSAMPLE)


# speedup vs baseline: 10.8581x; 2.0850x over previous
import numpy as np

import jax
import jax.numpy as jnp
from jax import lax
from jax.experimental import pallas as pl
from jax.experimental.pallas import tpu as pltpu
from jax.experimental.pallas import tpu_sc as plsc

B = 4
N = 8192
S = 1024
NSAMPLE = 32
C = 128
NCH = C + 3
RAD2 = 0.2 * 0.2

NC = 2
NSUB = 16
L = 16
NW = NC * NSUB
WPB = NW // B
QPW = S // WPB
CPW = (NCH + WPB - 1) // WPB

NWRD = N // 16
QC = S // 128
WC = 16
WPC = NWRD // WC


def _mesh():
    return plsc.VectorSubcoreMesh(
        core_axis_name="c", subcore_axis_name="s", num_cores=NC, num_subcores=NSUB
    )


def _mask_body(pxr, pyr, pzr, qx, qy, qz, out):
    w8 = (1 << lax.broadcasted_iota(jnp.int32, (8, 1), 0)).astype(jnp.float32)
    qxv = qx[0, 0]
    qyv = qy[0, 0]
    qzv = qz[0, 0]
    for wp in range(WPC):
        cks = []
        for h in range(2):
            t = wp * 2 + h
            dx = pxr[0, t] - qxv
            dy = pyr[0, t] - qyv
            dz = pzr[0, t] - qzv
            m = dx * dx + dy * dy + dz * dz < RAD2
            cks.append(jnp.sum(jnp.where(m, w8, 0.0), axis=0))
        out[0, 0, wp, :] = (cks[0] + 256.0 * cks[1]).astype(jnp.int32)


def _mask(pxr, pyr, pzr, qx, qy, qz):
    pspec = pl.BlockSpec((1, N // 128, 8, 1), lambda b, qc, wc: (b, wc, 0, 0))
    qspec = pl.BlockSpec((1, 1, 1, 128), lambda b, qc, wc: (b, qc, 0, 0))
    return pl.pallas_call(
        _mask_body,
        grid=(B, QC, WC),
        in_specs=[pspec, pspec, pspec, qspec, qspec, qspec],
        out_specs=pl.BlockSpec((1, 1, WPC, 128), lambda b, qc, wc: (b, qc, wc, 0)),
        out_shape=jax.ShapeDtypeStruct((B, QC, NWRD, 128), jnp.int32),
    )(pxr, pyr, pzr, qx, qy, qz)


def _sel_body(words, idx_out, wbuf, cand, obuf):
    w = lax.axis_index("s") * NC + lax.axis_index("c")
    b = w // WPB
    qc = w % WPB
    pltpu.sync_copy(words.at[b, qc], wbuf)
    lanes = lax.iota(jnp.int32, L)

    def per_q(qi, carry):
        qs = jnp.full((L,), qi, jnp.int32)

        def grp(g, cnt):
            wv = plsc.load_gather(wbuf, [g * L + lanes, qs])
            mv0 = jnp.where(jnp.logical_and(wv != 0, cnt < NSAMPLE), 1, 0)

            def cond(state):
                mv, cnt2 = state
                any_nz = plsc.all_reduce_population_count(mv == 1)[0]
                return jnp.logical_and(any_nz > 0, cnt2 < NSAMPLE)

            def expand(state):
                mv, cnt2 = state
                l = plsc.all_reduce_ffs(mv == 1)[0]
                widx = g * L + l
                wsp = plsc.load_gather(
                    wbuf, [jnp.full((L,), widx, jnp.int32), qs])
                bits = (wsp >> lanes) & 1
                ranks = cnt2 + plsc.cumsum(bits) - 1
                mm = jnp.logical_and(bits == 1, ranks < NSAMPLE)
                plsc.store_scatter(cand, [ranks], widx * L + lanes, mask=mm)
                cnt3 = cnt2 + jnp.sum(bits)
                return jnp.where(lanes == l, 0, mv), cnt3

            return lax.while_loop(cond, expand, (mv0, cnt))[1]

        cnt = lax.fori_loop(0, NWRD // L, grp, jnp.int32(0))
        first = cand[pl.ds(0, L)][0]
        fill = jnp.where(cnt > 0, first, 0)
        for h in range(NSAMPLE // L):
            slots = h * L + lanes
            cur = cand[pl.ds(h * L, L)]
            obuf[qi, pl.ds(h * L, L)] = jnp.where(slots < cnt, cur, fill)
        return carry

    lax.fori_loop(0, QPW, per_q, 0)
    pltpu.sync_copy(obuf, idx_out.at[b, pl.ds(qc * QPW, QPW)])


def _select(words):
    return pl.kernel(
        _sel_body,
        out_type=jax.ShapeDtypeStruct((B, S, NSAMPLE), jnp.int32),
        mesh=_mesh(),
        scratch_types=[
            pltpu.VMEM((NWRD, 128), jnp.int32),
            pltpu.VMEM((NSAMPLE,), jnp.int32),
            pltpu.VMEM((QPW, NSAMPLE), jnp.int32),
        ],
        compiler_params=pltpu.CompilerParams(needs_layout_passes=False),
    )(words)


def _grp_body(xt3, ft, nxt, idxf, out, ib, rb, ob, cb):
    w = lax.axis_index("s") * NC + lax.axis_index("c")
    b = w // WPB
    g = w % WPB
    pltpu.sync_copy(idxf.at[b], ib)
    pltpu.sync_copy(nxt.at[b], cb)
    nv = S * NSAMPLE // L

    def per_chan(j, carry):
        ch = g + WPB * j

        @pl.when(ch < 3)
        def _():
            pltpu.sync_copy(xt3.at[b * 3 + ch], rb)

        @pl.when(jnp.logical_and(ch >= 3, ch < NCH))
        def _():
            pltpu.sync_copy(ft.at[b * C + ch - 3], rb)

        @pl.when(ch < NCH)
        def _():
            zv = jnp.zeros((L,), jnp.int32)

            def gstep(v, c2):
                off = pl.multiple_of(v * L, L)
                ivec = ib[0, pl.ds(off, L)]
                ob[0, pl.ds(off, L)] = plsc.load_gather(rb, [zv, ivec])
                return c2

            lax.fori_loop(0, nv, gstep, 0)

            @pl.when(ch < 3)
            def _():
                crow = jnp.where(ch < 3, ch, 0)

                def sgrp(sg, c3):
                    soff = pl.multiple_of(sg * L, L)
                    ctrv = cb[crow, pl.ds(soff, L)]
                    for i in range(L):
                        ctr = ctrv[i]
                        p = (sg * L + i) * NSAMPLE
                        for h in range(NSAMPLE // L):
                            off = pl.multiple_of(p + h * L, L)
                            ob[0, pl.ds(off, L)] = ob[0, pl.ds(off, L)] - ctr
                    return c3

                lax.fori_loop(0, S // L, sgrp, 0)

            pltpu.sync_copy(ob, out.at[b * NCH + ch])

        return carry

    lax.fori_loop(0, CPW, per_chan, 0)


def _group(xt3, ft, nxt, idxf):
    return pl.kernel(
        _grp_body,
        out_type=jax.ShapeDtypeStruct((B * NCH, 1, S * NSAMPLE), jnp.float32),
        mesh=_mesh(),
        scratch_types=[
            pltpu.VMEM((1, S * NSAMPLE), jnp.int32),
            pltpu.VMEM((1, N), jnp.float32),
            pltpu.VMEM((1, S * NSAMPLE), jnp.float32),
            pltpu.VMEM((3, S), jnp.float32),
        ],
        compiler_params=pltpu.CompilerParams(needs_layout_passes=False),
    )(xt3, ft, nxt, idxf)


def kernel(xyz, new_xyz, features):
    pxr = xyz[..., 0].reshape(B, N // 8, 8, 1)
    pyr = xyz[..., 1].reshape(B, N // 8, 8, 1)
    pzr = xyz[..., 2].reshape(B, N // 8, 8, 1)
    qx = new_xyz[..., 0].reshape(B, QC, 1, 128)
    qy = new_xyz[..., 1].reshape(B, QC, 1, 128)
    qz = new_xyz[..., 2].reshape(B, QC, 1, 128)
    words = _mask(pxr, pyr, pzr, qx, qy, qz)
    idx = _select(words)
    idxf = idx.reshape(B, 1, S * NSAMPLE)
    xt3 = jnp.transpose(xyz, (0, 2, 1)).reshape(B * 3, 1, N)
    ft = features.reshape(B * C, 1, N)
    nxt = jnp.transpose(new_xyz, (0, 2, 1))
    out = _group(xt3, ft, nxt, idxf)
    return out.reshape(B, NCH, S, NSAMPLE)


# speedup vs baseline: 16.1017x; 1.4829x over previous
import numpy as np

import jax
import jax.numpy as jnp
from jax import lax
from jax.experimental import pallas as pl
from jax.experimental.pallas import tpu as pltpu
from jax.experimental.pallas import tpu_sc as plsc

B = 4
N = 8192
S = 1024
NSAMPLE = 32
C = 128
NCH = C + 3
RAD2 = 0.2 * 0.2

NC = 2
NSUB = 16
L = 16
NW = NC * NSUB
WPB = NW // B
QPW = S // WPB
CPW = (NCH + WPB - 1) // WPB

NWRD = N // 16
PCHK = 2048
NCHK = N // PCHK
QT = 128
NQT = S // QT

_WPACK = np.where(
    (np.arange(PCHK)[:, None] // 16) == np.arange(PCHK // 16)[None, :],
    np.exp2(np.arange(PCHK) % 16)[:, None],
    0.0,
).astype(np.float32)


def _mesh():
    return plsc.VectorSubcoreMesh(
        core_axis_name="c", subcore_axis_name="s", num_cores=NC, num_subcores=NSUB
    )


def _mask_body(px, py, pz, qxr, qyr, qzr, wmat, out, msk):
    qv = [(qxr[0, qs], qyr[0, qs], qzr[0, qs]) for qs in range(QT // 8)]
    for c in range(NCHK):
        for t in range(PCHK // 128):
            o = c * PCHK + t * 128
            pxv = px[0, 0, pl.ds(o, 128)][None, :]
            pyv = py[0, 0, pl.ds(o, 128)][None, :]
            pzv = pz[0, 0, pl.ds(o, 128)][None, :]
            for qs in range(QT // 8):
                qxv, qyv, qzv = qv[qs]
                dx = qxv - pxv
                dy = qyv - pyv
                dz = qzv - pzv
                d = dx * dx + dy * dy + dz * dz
                msk[qs * 8:(qs + 1) * 8, t * 128:(t + 1) * 128] = jnp.where(
                    d < RAD2, 1.0, 0.0)
        words = jnp.dot(msk[...], wmat[...],
                        preferred_element_type=jnp.float32)
        out[0, :, pl.ds(c * (PCHK // 16), PCHK // 16)] = words.astype(jnp.int32)


def _mask(px, py, pz, qxr, qyr, qzr, wmat):
    pspec = pl.BlockSpec((1, 1, N), lambda b, qt: (b, 0, 0))
    qspec = pl.BlockSpec((1, QT // 8, 8, 1), lambda b, qt: (b, qt, 0, 0))
    return pl.pallas_call(
        _mask_body,
        grid=(B, NQT),
        in_specs=[pspec, pspec, pspec, qspec, qspec, qspec,
                  pl.BlockSpec((PCHK, PCHK // 16), lambda b, qt: (0, 0))],
        out_specs=pl.BlockSpec((1, QT, NWRD), lambda b, qt: (b, qt, 0)),
        out_shape=jax.ShapeDtypeStruct((B, S, NWRD), jnp.int32),
        scratch_shapes=[pltpu.VMEM((QT, PCHK), jnp.float32)],
    )(px, py, pz, qxr, qyr, qzr, wmat)


def _sel_body(words, idx_out, wbuf, cand, obuf):
    w = lax.axis_index("s") * NC + lax.axis_index("c")
    b = w // WPB
    qc = w % WPB
    pltpu.sync_copy(words.at[b, pl.ds(qc * QPW, QPW)], wbuf)
    lanes = lax.iota(jnp.int32, L)

    def per_q(qi, carry):
        def grp(g, cnt):
            off = pl.multiple_of(g * L, L)
            wv = wbuf[qi, pl.ds(off, L)]
            mv0 = jnp.where(jnp.logical_and(wv != 0, cnt < NSAMPLE), 1, 0)

            def cond(state):
                mv, cnt2 = state
                any_nz = plsc.all_reduce_population_count(mv == 1)[0]
                return jnp.logical_and(any_nz > 0, cnt2 < NSAMPLE)

            def expand(state):
                mv, cnt2 = state
                l = plsc.all_reduce_ffs(mv == 1)[0]
                widx = g * L + l
                wsp = plsc.load_gather(
                    wbuf, [jnp.full((L,), qi, jnp.int32),
                           jnp.full((L,), widx, jnp.int32)])
                bits = (wsp >> lanes) & 1
                ranks = cnt2 + plsc.cumsum(bits) - 1
                mm = jnp.logical_and(bits == 1, ranks < NSAMPLE)
                plsc.store_scatter(cand, [ranks], widx * L + lanes, mask=mm)
                cnt3 = cnt2 + jnp.sum(bits)
                return jnp.where(lanes == l, 0, mv), cnt3

            return lax.while_loop(cond, expand, (mv0, cnt))[1]

        cnt = lax.fori_loop(0, NWRD // L, grp, jnp.int32(0))
        first = cand[pl.ds(0, L)][0]
        fill = jnp.where(cnt > 0, first, 0)
        for h in range(NSAMPLE // L):
            slots = h * L + lanes
            cur = cand[pl.ds(h * L, L)]
            off = pl.multiple_of(qi * NSAMPLE + h * L, L)
            obuf[0, pl.ds(off, L)] = jnp.where(slots < cnt, cur, fill)
        return carry

    lax.fori_loop(0, QPW, per_q, 0)
    pltpu.sync_copy(
        obuf, idx_out.at[b, :, pl.ds(qc * QPW * NSAMPLE, QPW * NSAMPLE)])


def _select(words):
    return pl.kernel(
        _sel_body,
        out_type=jax.ShapeDtypeStruct((B, 1, S * NSAMPLE), jnp.int32),
        mesh=_mesh(),
        scratch_types=[
            pltpu.VMEM((QPW, NWRD), jnp.int32),
            pltpu.VMEM((NSAMPLE,), jnp.int32),
            pltpu.VMEM((1, QPW * NSAMPLE), jnp.int32),
        ],
        compiler_params=pltpu.CompilerParams(needs_layout_passes=False),
    )(words)


def _grp_body(xt3, ft, nxt, idxf, out, ib, rb, ob, cb):
    w = lax.axis_index("s") * NC + lax.axis_index("c")
    b = w // WPB
    g = w % WPB
    pltpu.sync_copy(idxf.at[b], ib)
    pltpu.sync_copy(nxt.at[b], cb)
    nv = S * NSAMPLE // L

    def per_chan(j, carry):
        ch = g + WPB * j

        @pl.when(ch < 3)
        def _():
            pltpu.sync_copy(xt3.at[b * 3 + ch], rb)

        @pl.when(jnp.logical_and(ch >= 3, ch < NCH))
        def _():
            pltpu.sync_copy(ft.at[b * C + ch - 3], rb)

        @pl.when(ch < NCH)
        def _():
            zv = jnp.zeros((L,), jnp.int32)

            def gstep(v, c2):
                off = pl.multiple_of(v * L, L)
                ivec = ib[0, pl.ds(off, L)]
                ob[0, pl.ds(off, L)] = plsc.load_gather(rb, [zv, ivec])
                return c2

            lax.fori_loop(0, nv, gstep, 0)

            @pl.when(ch < 3)
            def _():
                crow = jnp.where(ch < 3, ch, 0)

                def sgrp(sg, c3):
                    soff = pl.multiple_of(sg * L, L)
                    ctrv = cb[crow, pl.ds(soff, L)]
                    for i in range(L):
                        ctr = ctrv[i]
                        p = (sg * L + i) * NSAMPLE
                        for h in range(NSAMPLE // L):
                            off = pl.multiple_of(p + h * L, L)
                            ob[0, pl.ds(off, L)] = ob[0, pl.ds(off, L)] - ctr
                    return c3

                lax.fori_loop(0, S // L, sgrp, 0)

            pltpu.sync_copy(ob, out.at[b * NCH + ch])

        return carry

    lax.fori_loop(0, CPW, per_chan, 0)


def _group(xt3, ft, nxt, idxf):
    return pl.kernel(
        _grp_body,
        out_type=jax.ShapeDtypeStruct((B * NCH, 1, S * NSAMPLE), jnp.float32),
        mesh=_mesh(),
        scratch_types=[
            pltpu.VMEM((1, S * NSAMPLE), jnp.int32),
            pltpu.VMEM((1, N), jnp.float32),
            pltpu.VMEM((1, S * NSAMPLE), jnp.float32),
            pltpu.VMEM((3, S), jnp.float32),
        ],
        compiler_params=pltpu.CompilerParams(needs_layout_passes=False),
    )(xt3, ft, nxt, idxf)


def kernel(xyz, new_xyz, features):
    px = xyz[..., 0].reshape(B, 1, N)
    py = xyz[..., 1].reshape(B, 1, N)
    pz = xyz[..., 2].reshape(B, 1, N)
    qxr = new_xyz[..., 0].reshape(B, S // 8, 8, 1)
    qyr = new_xyz[..., 1].reshape(B, S // 8, 8, 1)
    qzr = new_xyz[..., 2].reshape(B, S // 8, 8, 1)
    wmat = jnp.asarray(_WPACK)
    words = _mask(px, py, pz, qxr, qyr, qzr, wmat)
    idxf = _select(words)
    xt3 = jnp.transpose(xyz, (0, 2, 1)).reshape(B * 3, 1, N)
    ft = features.reshape(B * C, 1, N)
    nxt = jnp.transpose(new_xyz, (0, 2, 1))
    out = _group(xt3, ft, nxt, idxf)
    return out.reshape(B, NCH, S, NSAMPLE)
